# baseline (speedup 1.0000x reference)
import functools

import jax
import jax.numpy as jnp
from jax import lax
from jax.experimental import pallas as pl
from jax.experimental.pallas import tpu as pltpu
from jax.experimental.pallas import tpu_sc as plsc

NC = 2
NS = 16
NW = NC * NS

BATCH = 16384
D_GMF = 64
D_MLP = 256

B_PER_W = BATCH // NW
CHUNK = 128
N_CHUNKS = B_PER_W // CHUNK

TC_TILE = 1024


def _sc_mesh():
    return plsc.VectorSubcoreMesh(
        core_axis_name="c", subcore_axis_name="s", num_cores=NC,
        num_subcores=NS)


def _sc_gather_all(users, items, Wum, Wim, Wgmf):
    out_type = (
        jax.ShapeDtypeStruct((BATCH, D_MLP), jnp.float32),
        jax.ShapeDtypeStruct((BATCH, D_MLP), jnp.float32),
        jax.ShapeDtypeStruct((BATCH, 2 * D_GMF), jnp.float32),
        jax.ShapeDtypeStruct((BATCH, 2 * D_GMF), jnp.float32),
    )
    scratch_types = [
        pltpu.VMEM((B_PER_W,), jnp.int32),
        pltpu.VMEM((B_PER_W,), jnp.int32),
        pltpu.VMEM((CHUNK, D_MLP), jnp.float32),
        pltpu.VMEM((CHUNK, D_MLP), jnp.float32),
        pltpu.VMEM((CHUNK, 2 * D_GMF), jnp.float32),
        pltpu.VMEM((CHUNK, 2 * D_GMF), jnp.float32),
        pltpu.SemaphoreType.DMA,
        pltpu.SemaphoreType.DMA,
        pltpu.SemaphoreType.DMA,
        pltpu.SemaphoreType.DMA,
    ]

    @functools.partial(pl.kernel, out_type=out_type, mesh=_sc_mesh(),
                       scratch_types=scratch_types)
    def k(u_hbm, i_hbm, wum_hbm, wim_hbm, wg_hbm,
          eum_hbm, eim_hbm, gu_hbm, gi_hbm,
          idx_u, idx_i, um_v, im_v, gu_v, gi_v, s0, s1, s2, s3):
        wid = lax.axis_index("s") * NC + lax.axis_index("c")
        base = wid * B_PER_W
        pltpu.sync_copy(u_hbm.at[pl.ds(base, B_PER_W)], idx_u)
        pltpu.sync_copy(i_hbm.at[pl.ds(base, B_PER_W)], idx_i)
        for c in range(N_CHUNKS):
            iu = idx_u.at[pl.ds(c * CHUNK, CHUNK)]
            ii = idx_i.at[pl.ds(c * CHUNK, CHUNK)]
            c0 = pltpu.async_copy(wum_hbm.at[iu], um_v, s0)
            c1 = pltpu.async_copy(wim_hbm.at[ii], im_v, s1)
            c2 = pltpu.async_copy(wg_hbm.at[iu], gu_v, s2)
            c3 = pltpu.async_copy(wg_hbm.at[ii], gi_v, s3)
            c0.wait()
            c1.wait()
            c2.wait()
            c3.wait()
            row = base + c * CHUNK
            pltpu.sync_copy(um_v, eum_hbm.at[pl.ds(row, CHUNK)])
            pltpu.sync_copy(im_v, eim_hbm.at[pl.ds(row, CHUNK)])
            pltpu.sync_copy(gu_v, gu_hbm.at[pl.ds(row, CHUNK)])
            pltpu.sync_copy(gi_v, gi_hbm.at[pl.ds(row, CHUNK)])

    return k(users, items, Wum, Wim, Wgmf)


def _tc_mlp_body(eum_ref, eim_ref, gu_ref, gi_ref,
                 w1a_ref, w1b_ref, b1_ref, w2_ref, b2_ref, w3_ref, b3_ref,
                 wpg_ref, wpx_ref, bp_ref, out_ref):
    f32 = jnp.float32
    h1 = (jnp.dot(eum_ref[...], w1a_ref[...], preferred_element_type=f32)
          + jnp.dot(eim_ref[...], w1b_ref[...], preferred_element_type=f32)
          + b1_ref[...])
    h1 = jnp.maximum(h1, 0.0)
    h2 = jnp.maximum(
        jnp.dot(h1, w2_ref[...], preferred_element_type=f32) + b2_ref[...], 0.0)
    h3 = jnp.maximum(
        jnp.dot(h2, w3_ref[...], preferred_element_type=f32) + b3_ref[...], 0.0)
    g = gu_ref[:, :D_GMF] * gi_ref[:, D_GMF:]
    p = (jnp.dot(g, wpg_ref[...], preferred_element_type=f32)
         + jnp.dot(h3, wpx_ref[...], preferred_element_type=f32)
         + bp_ref[...])
    out_ref[...] = jax.nn.sigmoid(p[:, 0])


def _tc_mlp(eum, eim, gu, gi, w1a, w1b, b1, w2, b2, w3, b3, wpg, wpx, bp):
    n = eum.shape[0]
    grid = (n // TC_TILE,)
    full = lambda i: (0, 0)
    return pl.pallas_call(
        _tc_mlp_body,
        grid=grid,
        in_specs=[
            pl.BlockSpec((TC_TILE, D_MLP), lambda i: (i, 0)),
            pl.BlockSpec((TC_TILE, D_MLP), lambda i: (i, 0)),
            pl.BlockSpec((TC_TILE, 2 * D_GMF), lambda i: (i, 0)),
            pl.BlockSpec((TC_TILE, 2 * D_GMF), lambda i: (i, 0)),
            pl.BlockSpec((D_MLP, D_MLP), full),
            pl.BlockSpec((D_MLP, D_MLP), full),
            pl.BlockSpec((1, D_MLP), full),
            pl.BlockSpec((D_MLP, D_MLP // 2), full),
            pl.BlockSpec((1, D_MLP // 2), full),
            pl.BlockSpec((D_MLP // 2, D_GMF), full),
            pl.BlockSpec((1, D_GMF), full),
            pl.BlockSpec((D_GMF, 1), full),
            pl.BlockSpec((D_GMF, 1), full),
            pl.BlockSpec((1, 1), full),
        ],
        out_specs=pl.BlockSpec((TC_TILE,), lambda i: (i,)),
        out_shape=jax.ShapeDtypeStruct((n,), jnp.float32),
    )(eum, eim, gu, gi, w1a, w1b, b1, w2, b2, w3, b3, wpg, wpx, bp)


def kernel(users, items, Wug, Wig, Wum, Wim, W1, b1, W2, b2, W3, b3, Wp, bp):
    wgmf = jnp.concatenate([Wug, Wig], axis=1)
    eum, eim, gu, gi = _sc_gather_all(users, items, Wum, Wim, wgmf)
    w1a = W1[:D_MLP]
    w1b = W1[D_MLP:]
    wpg = Wp[:D_GMF]
    wpx = Wp[D_GMF:]
    out = _tc_mlp(eum, eim, gu, gi,
                  w1a, w1b, b1.reshape(1, -1),
                  W2, b2.reshape(1, -1), W3, b3.reshape(1, -1),
                  wpg, wpx, bp.reshape(1, 1))
    return out.reshape(-1, 1)


# speedup vs baseline: 1.3390x; 1.3390x over previous
import functools

import jax
import jax.numpy as jnp
from jax import lax
from jax.experimental import pallas as pl
from jax.experimental.pallas import tpu as pltpu
from jax.experimental.pallas import tpu_sc as plsc

NC = 2
NS = 16
NW = NC * NS

BATCH = 16384
D_GMF = 64
D_MLP = 256

B_PER_W = BATCH // NW
CHUNK = 128
N_CHUNKS = B_PER_W // CHUNK

TC_TILE = 2048
TR_TILE = 4096


def _sc_mesh():
    return plsc.VectorSubcoreMesh(
        core_axis_name="c", subcore_axis_name="s", num_cores=NC,
        num_subcores=NS)


def _sc_gather_all(users, items, Wum, Wim, Wgmf):
    out_type = (
        jax.ShapeDtypeStruct((BATCH, D_MLP), jnp.float32),
        jax.ShapeDtypeStruct((BATCH, D_MLP), jnp.float32),
        jax.ShapeDtypeStruct((BATCH, 2 * D_GMF), jnp.float32),
        jax.ShapeDtypeStruct((BATCH, 2 * D_GMF), jnp.float32),
    )
    scratch_types = [
        pltpu.VMEM((B_PER_W,), jnp.int32),
        pltpu.VMEM((B_PER_W,), jnp.int32),
        pltpu.VMEM((CHUNK, D_MLP), jnp.float32),
        pltpu.VMEM((CHUNK, D_MLP), jnp.float32),
        pltpu.VMEM((CHUNK, 2 * D_GMF), jnp.float32),
        pltpu.VMEM((CHUNK, 2 * D_GMF), jnp.float32),
        pltpu.SemaphoreType.DMA,
        pltpu.SemaphoreType.DMA,
        pltpu.SemaphoreType.DMA,
        pltpu.SemaphoreType.DMA,
    ]

    @functools.partial(pl.kernel, out_type=out_type, mesh=_sc_mesh(),
                       scratch_types=scratch_types)
    def k(u_hbm, i_hbm, wum_hbm, wim_hbm, wg_hbm,
          eum_hbm, eim_hbm, gu_hbm, gi_hbm,
          idx_u, idx_i, um_v, im_v, gu_v, gi_v, s0, s1, s2, s3):
        wid = lax.axis_index("s") * NC + lax.axis_index("c")
        base = wid * B_PER_W
        pltpu.sync_copy(u_hbm.at[pl.ds(base, B_PER_W)], idx_u)
        pltpu.sync_copy(i_hbm.at[pl.ds(base, B_PER_W)], idx_i)
        for c in range(N_CHUNKS):
            iu = idx_u.at[pl.ds(c * CHUNK, CHUNK)]
            ii = idx_i.at[pl.ds(c * CHUNK, CHUNK)]
            c0 = pltpu.async_copy(wum_hbm.at[iu], um_v, s0)
            c1 = pltpu.async_copy(wim_hbm.at[ii], im_v, s1)
            c2 = pltpu.async_copy(wg_hbm.at[iu], gu_v, s2)
            c3 = pltpu.async_copy(wg_hbm.at[ii], gi_v, s3)
            c0.wait()
            c1.wait()
            c2.wait()
            c3.wait()
            row = base + c * CHUNK
            pltpu.sync_copy(um_v, eum_hbm.at[pl.ds(row, CHUNK)])
            pltpu.sync_copy(im_v, eim_hbm.at[pl.ds(row, CHUNK)])
            pltpu.sync_copy(gu_v, gu_hbm.at[pl.ds(row, CHUNK)])
            pltpu.sync_copy(gi_v, gi_hbm.at[pl.ds(row, CHUNK)])

    return k(users, items, Wum, Wim, Wgmf)


def _trc_body(at_ref, bt_ref, o_ref):
    o_ref[:, :D_GMF] = jnp.transpose(at_ref[...], (1, 0))
    o_ref[:, D_GMF:] = jnp.transpose(bt_ref[...], (1, 0))


def _tc_transpose_concat(WugT, WigT):
    n = WugT.shape[1]
    grid = (pl.cdiv(n, TR_TILE),)
    return pl.pallas_call(
        _trc_body,
        grid=grid,
        in_specs=[
            pl.BlockSpec((D_GMF, TR_TILE), lambda i: (0, i)),
            pl.BlockSpec((D_GMF, TR_TILE), lambda i: (0, i)),
        ],
        out_specs=pl.BlockSpec((TR_TILE, 2 * D_GMF), lambda i: (i, 0)),
        out_shape=jax.ShapeDtypeStruct((n, 2 * D_GMF), jnp.float32),
    )(WugT, WigT)


def _tc_mlp_body(eum_ref, eim_ref, gu_ref, gi_ref,
                 w1a_ref, w1b_ref, b1_ref, w2_ref, b2_ref, w3_ref, b3_ref,
                 wpg_ref, wpx_ref, bp_ref, out_ref):
    f32 = jnp.float32
    h1 = (jnp.dot(eum_ref[...], w1a_ref[...], preferred_element_type=f32)
          + jnp.dot(eim_ref[...], w1b_ref[...], preferred_element_type=f32)
          + b1_ref[...])
    h1 = jnp.maximum(h1, 0.0)
    h2 = jnp.maximum(
        jnp.dot(h1, w2_ref[...], preferred_element_type=f32) + b2_ref[...], 0.0)
    h3 = jnp.maximum(
        jnp.dot(h2, w3_ref[...], preferred_element_type=f32) + b3_ref[...], 0.0)
    g = gu_ref[:, :D_GMF] * gi_ref[:, D_GMF:]
    p = (jnp.dot(g, wpg_ref[...], preferred_element_type=f32)
         + jnp.dot(h3, wpx_ref[...], preferred_element_type=f32)
         + bp_ref[...])
    out_ref[...] = jax.nn.sigmoid(p[:, 0])


def _tc_mlp(eum, eim, gu, gi, w1a, w1b, b1, w2, b2, w3, b3, wpg, wpx, bp):
    n = eum.shape[0]
    grid = (n // TC_TILE,)
    full = lambda i: (0, 0)
    return pl.pallas_call(
        _tc_mlp_body,
        grid=grid,
        in_specs=[
            pl.BlockSpec((TC_TILE, D_MLP), lambda i: (i, 0)),
            pl.BlockSpec((TC_TILE, D_MLP), lambda i: (i, 0)),
            pl.BlockSpec((TC_TILE, 2 * D_GMF), lambda i: (i, 0)),
            pl.BlockSpec((TC_TILE, 2 * D_GMF), lambda i: (i, 0)),
            pl.BlockSpec((D_MLP, D_MLP), full),
            pl.BlockSpec((D_MLP, D_MLP), full),
            pl.BlockSpec((1, D_MLP), full),
            pl.BlockSpec((D_MLP, D_MLP // 2), full),
            pl.BlockSpec((1, D_MLP // 2), full),
            pl.BlockSpec((D_MLP // 2, D_GMF), full),
            pl.BlockSpec((1, D_GMF), full),
            pl.BlockSpec((D_GMF, 1), full),
            pl.BlockSpec((D_GMF, 1), full),
            pl.BlockSpec((1, 1), full),
        ],
        out_specs=pl.BlockSpec((TC_TILE,), lambda i: (i,)),
        out_shape=jax.ShapeDtypeStruct((n,), jnp.float32),
    )(eum, eim, gu, gi, w1a, w1b, b1, w2, b2, w3, b3, wpg, wpx, bp)


def kernel(users, items, Wug, Wig, Wum, Wim, W1, b1, W2, b2, W3, b3, Wp, bp):
    wgmf = _tc_transpose_concat(Wug.T, Wig.T)
    eum, eim, gu, gi = _sc_gather_all(users, items, Wum, Wim, wgmf)
    w1a = W1[:D_MLP]
    w1b = W1[D_MLP:]
    wpg = Wp[:D_GMF]
    wpx = Wp[D_GMF:]
    out = _tc_mlp(eum, eim, gu, gi,
                  w1a, w1b, b1.reshape(1, -1),
                  W2, b2.reshape(1, -1), W3, b3.reshape(1, -1),
                  wpg, wpx, bp.reshape(1, 1))
    return out.reshape(-1, 1)


# speedup vs baseline: 1.3826x; 1.0326x over previous
import functools

import jax
import jax.numpy as jnp
from jax import lax
from jax.experimental import pallas as pl
from jax.experimental.pallas import tpu as pltpu
from jax.experimental.pallas import tpu_sc as plsc

NC = 2
NS = 16
NW = NC * NS

BATCH = 16384
D_GMF = 64
D_MLP = 256

B_PER_W = BATCH // NW
CHUNK = 128
N_CHUNKS = B_PER_W // CHUNK

TC_TILE = 2048
TR_TILE = 8192


def _sc_mesh():
    return plsc.VectorSubcoreMesh(
        core_axis_name="c", subcore_axis_name="s", num_cores=NC,
        num_subcores=NS)


def _sc_gather_all(users, items, Wum, Wim, Wgmf):
    out_type = (
        jax.ShapeDtypeStruct((BATCH, D_MLP), jnp.float32),
        jax.ShapeDtypeStruct((BATCH, D_MLP), jnp.float32),
        jax.ShapeDtypeStruct((BATCH, 2 * D_GMF), jnp.float32),
        jax.ShapeDtypeStruct((BATCH, 2 * D_GMF), jnp.float32),
    )
    scratch_types = [
        pltpu.VMEM((B_PER_W,), jnp.int32),
        pltpu.VMEM((B_PER_W,), jnp.int32),
        pltpu.VMEM((CHUNK, D_MLP), jnp.float32),
        pltpu.VMEM((CHUNK, D_MLP), jnp.float32),
        pltpu.VMEM((CHUNK, 2 * D_GMF), jnp.float32),
        pltpu.VMEM((CHUNK, 2 * D_GMF), jnp.float32),
        pltpu.SemaphoreType.DMA,
        pltpu.SemaphoreType.DMA,
        pltpu.SemaphoreType.DMA,
        pltpu.SemaphoreType.DMA,
    ]

    @functools.partial(pl.kernel, out_type=out_type, mesh=_sc_mesh(),
                       scratch_types=scratch_types)
    def k(u_hbm, i_hbm, wum_hbm, wim_hbm, wg_hbm,
          eum_hbm, eim_hbm, gu_hbm, gi_hbm,
          idx_u, idx_i, um_v, im_v, gu_v, gi_v, s0, s1, s2, s3):
        wid = lax.axis_index("s") * NC + lax.axis_index("c")
        base = wid * B_PER_W
        pltpu.sync_copy(u_hbm.at[pl.ds(base, B_PER_W)], idx_u)
        pltpu.sync_copy(i_hbm.at[pl.ds(base, B_PER_W)], idx_i)
        for c in range(N_CHUNKS):
            iu = idx_u.at[pl.ds(c * CHUNK, CHUNK)]
            ii = idx_i.at[pl.ds(c * CHUNK, CHUNK)]
            c0 = pltpu.async_copy(wum_hbm.at[iu], um_v, s0)
            c1 = pltpu.async_copy(wim_hbm.at[ii], im_v, s1)
            c2 = pltpu.async_copy(wg_hbm.at[iu], gu_v, s2)
            c3 = pltpu.async_copy(wg_hbm.at[ii], gi_v, s3)
            c0.wait()
            c1.wait()
            c2.wait()
            c3.wait()
            row = base + c * CHUNK
            pltpu.sync_copy(um_v, eum_hbm.at[pl.ds(row, CHUNK)])
            pltpu.sync_copy(im_v, eim_hbm.at[pl.ds(row, CHUNK)])
            pltpu.sync_copy(gu_v, gu_hbm.at[pl.ds(row, CHUNK)])
            pltpu.sync_copy(gi_v, gi_hbm.at[pl.ds(row, CHUNK)])

    return k(users, items, Wum, Wim, Wgmf)


def _trc_body(at_ref, bt_ref, o_ref):
    eye = jnp.eye(D_GMF, dtype=jnp.float32)
    dn = (((0,), (0,)), ((), ()))
    o_ref[:, :D_GMF] = lax.dot_general(at_ref[...], eye, dn,
                                       preferred_element_type=jnp.float32)
    o_ref[:, D_GMF:] = lax.dot_general(bt_ref[...], eye, dn,
                                       preferred_element_type=jnp.float32)


def _tc_transpose_concat(WugT, WigT):
    n = WugT.shape[1]
    grid = (pl.cdiv(n, TR_TILE),)
    return pl.pallas_call(
        _trc_body,
        grid=grid,
        compiler_params=pltpu.CompilerParams(
            fuse_transposed_lhs_in_matmul=True),
        in_specs=[
            pl.BlockSpec((D_GMF, TR_TILE), lambda i: (0, i)),
            pl.BlockSpec((D_GMF, TR_TILE), lambda i: (0, i)),
        ],
        out_specs=pl.BlockSpec((TR_TILE, 2 * D_GMF), lambda i: (i, 0)),
        out_shape=jax.ShapeDtypeStruct((n, 2 * D_GMF), jnp.float32),
    )(WugT, WigT)


def _tc_mlp_body(eum_ref, eim_ref, gu_ref, gi_ref,
                 w1a_ref, w1b_ref, b1_ref, w2_ref, b2_ref, w3_ref, b3_ref,
                 wpg_ref, wpx_ref, bp_ref, out_ref):
    f32, bf16 = jnp.float32, jnp.bfloat16
    h1 = (jnp.dot(eum_ref[...].astype(bf16), w1a_ref[...].astype(bf16),
                  preferred_element_type=f32)
          + jnp.dot(eim_ref[...].astype(bf16), w1b_ref[...].astype(bf16),
                    preferred_element_type=f32)
          + b1_ref[...])
    h1 = jnp.maximum(h1, 0.0).astype(bf16)
    h2 = jnp.maximum(
        jnp.dot(h1, w2_ref[...].astype(bf16), preferred_element_type=f32)
        + b2_ref[...], 0.0).astype(bf16)
    h3 = jnp.maximum(
        jnp.dot(h2, w3_ref[...].astype(bf16), preferred_element_type=f32)
        + b3_ref[...], 0.0).astype(bf16)
    g = (gu_ref[:, :D_GMF] * gi_ref[:, D_GMF:]).astype(bf16)
    p = (jnp.dot(g, wpg_ref[...].astype(bf16), preferred_element_type=f32)
         + jnp.dot(h3, wpx_ref[...].astype(bf16), preferred_element_type=f32)
         + bp_ref[...])
    out_ref[...] = jax.nn.sigmoid(p[:, 0])


def _tc_mlp(eum, eim, gu, gi, w1a, w1b, b1, w2, b2, w3, b3, wpg, wpx, bp):
    n = eum.shape[0]
    grid = (n // TC_TILE,)
    full = lambda i: (0, 0)
    return pl.pallas_call(
        _tc_mlp_body,
        grid=grid,
        in_specs=[
            pl.BlockSpec((TC_TILE, D_MLP), lambda i: (i, 0)),
            pl.BlockSpec((TC_TILE, D_MLP), lambda i: (i, 0)),
            pl.BlockSpec((TC_TILE, 2 * D_GMF), lambda i: (i, 0)),
            pl.BlockSpec((TC_TILE, 2 * D_GMF), lambda i: (i, 0)),
            pl.BlockSpec((D_MLP, D_MLP), full),
            pl.BlockSpec((D_MLP, D_MLP), full),
            pl.BlockSpec((1, D_MLP), full),
            pl.BlockSpec((D_MLP, D_MLP // 2), full),
            pl.BlockSpec((1, D_MLP // 2), full),
            pl.BlockSpec((D_MLP // 2, D_GMF), full),
            pl.BlockSpec((1, D_GMF), full),
            pl.BlockSpec((D_GMF, 1), full),
            pl.BlockSpec((D_GMF, 1), full),
            pl.BlockSpec((1, 1), full),
        ],
        out_specs=pl.BlockSpec((TC_TILE,), lambda i: (i,)),
        out_shape=jax.ShapeDtypeStruct((n,), jnp.float32),
    )(eum, eim, gu, gi, w1a, w1b, b1, w2, b2, w3, b3, wpg, wpx, bp)


def kernel(users, items, Wug, Wig, Wum, Wim, W1, b1, W2, b2, W3, b3, Wp, bp):
    wgmf = _tc_transpose_concat(Wug.T, Wig.T)
    eum, eim, gu, gi = _sc_gather_all(users, items, Wum, Wim, wgmf)
    w1a = W1[:D_MLP]
    w1b = W1[D_MLP:]
    wpg = Wp[:D_GMF]
    wpx = Wp[D_GMF:]
    out = _tc_mlp(eum, eim, gu, gi,
                  w1a, w1b, b1.reshape(1, -1),
                  W2, b2.reshape(1, -1), W3, b3.reshape(1, -1),
                  wpg, wpx, bp.reshape(1, 1))
    return out.reshape(-1, 1)
